# baseline (speedup 1.0000x reference)
import functools

import jax
import jax.numpy as jnp
from jax import lax
from jax.experimental import pallas as pl
from jax.experimental.pallas import tpu as pltpu
from jax.experimental.pallas import tpu_sc as plsc

_FEATURE_SIZE = 1000000
_EMBED = 16
_BATCH = 16384
_FEATS = 26
_TOTAL = _BATCH * _FEATS

_NC, _NS = 2, 16
_NW = _NC * _NS
_CHUNK = 1664
_CPW = _TOTAL // (_NW * _CHUNK)
_NBUF = 3


def _gather_body(idx_hbm, table_hbm, out_hbm, *scratch):
    idx_bufs = scratch[0:_NBUF]
    row_bufs = scratch[_NBUF:2 * _NBUF]
    gsems = scratch[2 * _NBUF:3 * _NBUF]
    wsems = scratch[3 * _NBUF:4 * _NBUF]
    wid = lax.axis_index("s") * _NC + lax.axis_index("c")

    def chunk_base(j):
        return (wid * _CPW + j) * _CHUNK

    def stage(j):
        b = j % _NBUF
        pltpu.sync_copy(idx_hbm.at[pl.ds(chunk_base(j), _CHUNK)], idx_bufs[b])
        pltpu.async_copy(table_hbm.at[idx_bufs[b]], row_bufs[b], gsems[b])

    for j in range(_NBUF - 1):
        stage(j)

    for j in range(_CPW):
        b = j % _NBUF
        nj = j + _NBUF - 1
        if nj < _CPW:
            nb = nj % _NBUF
            if nj >= _NBUF:
                pltpu.make_async_copy(
                    row_bufs[nb],
                    out_hbm.at[pl.ds(chunk_base(nj - _NBUF), _CHUNK)],
                    wsems[nb],
                ).wait()
            stage(nj)
        pltpu.make_async_copy(
            table_hbm.at[idx_bufs[b]], row_bufs[b], gsems[b]
        ).wait()
        pltpu.async_copy(
            row_bufs[b], out_hbm.at[pl.ds(chunk_base(j), _CHUNK)], wsems[b]
        )

    for j in range(max(_CPW - _NBUF, 0), _CPW):
        b = j % _NBUF
        pltpu.make_async_copy(
            row_bufs[b], out_hbm.at[pl.ds(chunk_base(j), _CHUNK)], wsems[b]
        ).wait()


@jax.jit
def _gather(idx_flat, table):
    mesh = plsc.VectorSubcoreMesh(core_axis_name="c", subcore_axis_name="s")
    run = functools.partial(
        pl.kernel,
        mesh=mesh,
        out_type=jax.ShapeDtypeStruct((_TOTAL, _EMBED), jnp.float32),
        compiler_params=pltpu.CompilerParams(use_tc_tiling_on_sc=False),
        scratch_types=(
            [pltpu.VMEM((_CHUNK,), jnp.int32) for _ in range(_NBUF)]
            + [pltpu.VMEM((_CHUNK, _EMBED), jnp.float32) for _ in range(_NBUF)]
            + [pltpu.SemaphoreType.DMA for _ in range(2 * _NBUF)]
        ),
    )(_gather_body)
    return run(idx_flat, table)


def kernel(inputs, table):
    out_flat = _gather(inputs.reshape(_TOTAL), table)
    return out_flat.reshape(_BATCH, _FEATS, _EMBED)


# speedup vs baseline: 1.3830x; 1.3830x over previous
import functools

import jax
import jax.numpy as jnp
from jax import lax
from jax.experimental import pallas as pl
from jax.experimental.pallas import tpu as pltpu
from jax.experimental.pallas import tpu_sc as plsc

_V = 1000000
_E = 16
_B = 16384
_F = 26
_TOTAL = _B * _F

_NC, _NS = 2, 16
_NW = _NC * _NS

_AU = 512
_NFULL = _V // _AU
_ATAIL = _V - _NFULL * _AU
_APW = 62

_C = 512
_CHUNKS = _TOTAL // _C
_CPW = _CHUNKS // _NW
_HPF = _B // _C


def _transpose_body(tab_t, tail_flat, scratch, s_in0, s_in1, s_out0, s_out1,
                    sin_sem0, sin_sem1, sout_sem0, sout_sem1):
    wid = lax.axis_index("s") * _NC + lax.axis_index("c")
    i16 = lax.iota(jnp.int32, 16) * 16
    s_in = (s_in0, s_in1)
    s_out = (s_out0, s_out1)
    sin_sem = (sin_sem0, sin_sem1)
    sout_sem = (sout_sem0, sout_sem1)

    def col0_of(ug):
        return jnp.minimum(ug, _NFULL - 1) * _AU

    def start_in(ug, b):
        pltpu.async_copy(
            tab_t.at[:, pl.ds(col0_of(ug), _AU)], s_in[b], sin_sem[b]
        )

    def wait_in(ug, b):
        pltpu.make_async_copy(
            tab_t.at[:, pl.ds(col0_of(ug), _AU)], s_in[b], sin_sem[b]
        ).wait()

    def start_out(ug, b):
        pltpu.async_copy(
            s_out[b], scratch.at[pl.ds(col0_of(ug) * _E, _AU * _E)],
            sout_sem[b],
        )

    def wait_out(ug, b):
        pltpu.make_async_copy(
            s_out[b], scratch.at[pl.ds(col0_of(ug) * _E, _AU * _E)],
            sout_sem[b],
        ).wait()

    def do_unit(ug, b):
        wait_in(ug, b)
        for d in range(_E):
            for k in range(_AU // 16):
                v = s_in[b][d, pl.ds(16 * k, 16)]
                plsc.store_scatter(s_out[b], [i16 + (256 * k + d)], v)
        start_out(ug, b)

    base = wid * _APW
    start_in(base, 0)
    start_in(base + 1, 1)

    def step(s, carry):
        u0 = base + 2 * s
        u1 = u0 + 1

        @pl.when(s > 0)
        def _():
            wait_out(u0 - 2, 0)

        do_unit(u0, 0)
        start_in(u0 + 2, 0)

        @pl.when(s > 0)
        def _():
            wait_out(u1 - 2, 1)

        do_unit(u1, 1)
        start_in(u1 + 2, 1)
        return carry

    lax.fori_loop(0, _APW // 2, step, 0)
    wait_in(base + _APW, 0)
    wait_in(base + _APW + 1, 1)
    wait_out(base + _APW - 2, 0)
    wait_out(base + _APW - 1, 1)

    @pl.when(wid == _NW - 1)
    def _():
        pltpu.sync_copy(tail_flat, s_out[0].at[pl.ds(0, _ATAIL * _E)])
        pltpu.sync_copy(s_out[0].at[pl.ds(0, _ATAIL * _E)],
                        scratch.at[pl.ds(_NFULL * _AU * _E, _ATAIL * _E)])


_R2C = 1664
_R2CPW = _TOTAL // (_NW * _R2C)
_NBUF = 3


def _rows_body(idx_hbm, table_hbm, out_hbm, *scratch):
    idx_bufs = scratch[0:_NBUF]
    row_bufs = scratch[_NBUF:2 * _NBUF]
    gsems = scratch[2 * _NBUF:3 * _NBUF]
    wsems = scratch[3 * _NBUF:4 * _NBUF]
    wid = lax.axis_index("s") * _NC + lax.axis_index("c")

    def chunk_base(j):
        return (wid * _R2CPW + j) * _R2C

    def stage(j):
        b = j % _NBUF
        pltpu.sync_copy(idx_hbm.at[pl.ds(chunk_base(j), _R2C)], idx_bufs[b])
        pltpu.async_copy(table_hbm.at[idx_bufs[b]], row_bufs[b], gsems[b])

    for j in range(_NBUF - 1):
        stage(j)

    for j in range(_R2CPW):
        b = j % _NBUF
        nj = j + _NBUF - 1
        if nj < _R2CPW:
            nb = nj % _NBUF
            if nj >= _NBUF:
                pltpu.make_async_copy(
                    row_bufs[nb],
                    out_hbm.at[pl.ds(chunk_base(nj - _NBUF), _R2C)],
                    wsems[nb],
                ).wait()
            stage(nj)
        pltpu.make_async_copy(
            table_hbm.at[idx_bufs[b]], row_bufs[b], gsems[b]
        ).wait()
        pltpu.async_copy(
            row_bufs[b], out_hbm.at[pl.ds(chunk_base(j), _R2C)], wsems[b]
        )

    for j in range(max(_R2CPW - _NBUF, 0), _R2CPW):
        b = j % _NBUF
        pltpu.make_async_copy(
            row_bufs[b], out_hbm.at[pl.ds(chunk_base(j), _R2C)], wsems[b]
        ).wait()


def _gather_body(idx_hbm, table2d, out_hbm, idx0, idx1, rows0, rows1,
                 ov0, ov1, g_sem0, g_sem1, o_sem0, o_sem1):
    wid = lax.axis_index("s") * _NC + lax.axis_index("c")
    i1 = lax.iota(jnp.int32, 16)
    idx_v = (idx0, idx1)
    rows_v = (rows0, rows1)
    ov = (ov0, ov1)
    g_sem = (g_sem0, g_sem1)
    o_sem = (o_sem0, o_sem1)

    def chunk_of(lq):
        g = wid * _CPW + jnp.minimum(lq, _CPW - 1)
        f = g // _HPF
        h = g % _HPF
        return f * _B + h * _C, f * (2 * _B * 8) + h * (_C * 8)

    def fetch(lq, b):
        src, _ = chunk_of(lq)
        pltpu.sync_copy(idx_hbm.at[pl.ds(src, _C)], idx_v[b])
        pltpu.async_copy(table2d.at[idx_v[b]], rows_v[b], g_sem[b])

    def wait_gather(b):
        pltpu.make_async_copy(table2d.at[idx_v[b]], rows_v[b], g_sem[b]).wait()

    def out_copy(lq, b, dt):
        _, dst = chunk_of(lq)
        return pltpu.make_async_copy(
            ov[b].at[pl.ds(dt * (_C * 8), _C * 8)],
            out_hbm.at[pl.ds(dst + dt * (_B * 8), _C * 8)],
            o_sem[b],
        )

    def do_chunk(lq, s, b):
        wait_gather(b)

        @pl.when(s > 0)
        def _():
            out_copy(lq - 2, b, 0).wait()
            out_copy(lq - 2, b, 1).wait()

        for dt in range(2):
            for ds in range(8):
                d = dt * 8 + ds
                ci = jnp.full((16,), d, jnp.int32)
                for btl in range(_C // 128):
                    for k in range(8):
                        v = plsc.load_gather(
                            rows_v[b], [i1 + (btl * 128 + 16 * k), ci]
                        )
                        ov[b][pl.ds(dt * (_C * 8) + btl * 1024 + ds * 128
                                    + 16 * k, 16)] = v
        out_copy(lq, b, 0).start()
        out_copy(lq, b, 1).start()
        fetch(lq + 2, b)

    fetch(0, 0)
    fetch(1, 1)

    def step(s, carry):
        do_chunk(2 * s, s, 0)
        do_chunk(2 * s + 1, s, 1)
        return carry

    lax.fori_loop(0, _CPW // 2, step, 0)
    wait_gather(0)
    wait_gather(1)
    out_copy(_CPW - 2, 0, 0).wait()
    out_copy(_CPW - 2, 0, 1).wait()
    out_copy(_CPW - 1, 1, 0).wait()
    out_copy(_CPW - 1, 1, 1).wait()


@jax.jit
def _embed(inputs, table):
    mesh = plsc.VectorSubcoreMesh(core_axis_name="c", subcore_axis_name="s")
    transpose_call = functools.partial(
        pl.kernel,
        mesh=mesh,
        out_type=jax.ShapeDtypeStruct((_V * _E,), jnp.float32),
        compiler_params=pltpu.CompilerParams(needs_layout_passes=False),
        scratch_types=(
            [pltpu.VMEM((_E, _AU), jnp.float32) for _ in range(2)]
            + [pltpu.VMEM((_AU * _E,), jnp.float32) for _ in range(2)]
            + [pltpu.SemaphoreType.DMA for _ in range(4)]
        ),
    )(_transpose_body)
    tail_flat = table[_NFULL * _AU:, :].reshape(_ATAIL * _E)
    scratch = transpose_call(table.T, tail_flat)

    gather_call = functools.partial(
        pl.kernel,
        mesh=mesh,
        out_type=jax.ShapeDtypeStruct((_F * _E * _B,), jnp.float32),
        compiler_params=pltpu.CompilerParams(
            use_tc_tiling_on_sc=False, needs_layout_passes=False
        ),
        scratch_types=(
            [pltpu.VMEM((_C,), jnp.int32) for _ in range(2)]
            + [pltpu.VMEM((_C, _E), jnp.float32) for _ in range(2)]
            + [pltpu.VMEM((_C * _E,), jnp.float32) for _ in range(2)]
            + [pltpu.SemaphoreType.DMA for _ in range(4)]
        ),
    )(_gather_body)
    rows_call = functools.partial(
        pl.kernel,
        mesh=mesh,
        out_type=jax.ShapeDtypeStruct((_TOTAL, _E), jnp.float32),
        compiler_params=pltpu.CompilerParams(use_tc_tiling_on_sc=False),
        scratch_types=(
            [pltpu.VMEM((_R2C,), jnp.int32) for _ in range(_NBUF)]
            + [pltpu.VMEM((_R2C, _E), jnp.float32) for _ in range(_NBUF)]
            + [pltpu.SemaphoreType.DMA for _ in range(2 * _NBUF)]
        ),
    )(_rows_body)
    out_rows = rows_call(inputs.reshape(_TOTAL), scratch.reshape(_V, _E))
    return out_rows.reshape(_B, _F, _E)


def kernel(inputs, table):
    return _embed(inputs, table)


# speedup vs baseline: 2.7728x; 2.0049x over previous
import functools

import jax
import jax.numpy as jnp
from jax import lax
from jax.experimental import pallas as pl
from jax.experimental.pallas import tpu as pltpu
from jax.experimental.pallas import tpu_sc as plsc

_V = 1000000
_E = 16
_B = 16384
_F = 26
_TOTAL = _B * _F

_NC, _NS = 2, 16
_NW = _NC * _NS

_AU = 512
_NFULL = _V // _AU
_ATAIL = _V - _NFULL * _AU
_APW = 62

_C = 512
_CHUNKS = _TOTAL // _C
_CPW = _CHUNKS // _NW
_HPF = _B // _C


def _transpose_body(tab_t, tail_flat, scratch, s_in0, s_in1, s_out0, s_out1,
                    sin_sem0, sin_sem1, sout_sem0, sout_sem1):
    wid = lax.axis_index("s") * _NC + lax.axis_index("c")
    i16 = lax.iota(jnp.int32, 16) * 16
    s_in = (s_in0, s_in1)
    s_out = (s_out0, s_out1)
    sin_sem = (sin_sem0, sin_sem1)
    sout_sem = (sout_sem0, sout_sem1)

    def col0_of(ug):
        return jnp.minimum(ug, _NFULL - 1) * _AU

    def start_in(ug, b):
        pltpu.async_copy(
            tab_t.at[:, pl.ds(col0_of(ug), _AU)], s_in[b], sin_sem[b]
        )

    def wait_in(ug, b):
        pltpu.make_async_copy(
            tab_t.at[:, pl.ds(col0_of(ug), _AU)], s_in[b], sin_sem[b]
        ).wait()

    def start_out(ug, b):
        pltpu.async_copy(
            s_out[b], scratch.at[pl.ds(col0_of(ug) * _E, _AU * _E)],
            sout_sem[b],
        )

    def wait_out(ug, b):
        pltpu.make_async_copy(
            s_out[b], scratch.at[pl.ds(col0_of(ug) * _E, _AU * _E)],
            sout_sem[b],
        ).wait()

    def do_unit(ug, b):
        wait_in(ug, b)
        for d in range(_E):
            for k in range(_AU // 16):
                v = s_in[b][d, pl.ds(16 * k, 16)]
                plsc.store_scatter(s_out[b], [i16 + (256 * k + d)], v)
        start_out(ug, b)

    base = wid * _APW
    start_in(base, 0)
    start_in(base + 1, 1)

    def step(s, carry):
        u0 = base + 2 * s
        u1 = u0 + 1

        @pl.when(s > 0)
        def _():
            wait_out(u0 - 2, 0)

        do_unit(u0, 0)
        start_in(u0 + 2, 0)

        @pl.when(s > 0)
        def _():
            wait_out(u1 - 2, 1)

        do_unit(u1, 1)
        start_in(u1 + 2, 1)
        return carry

    lax.fori_loop(0, _APW // 2, step, 0)
    wait_in(base + _APW, 0)
    wait_in(base + _APW + 1, 1)
    wait_out(base + _APW - 2, 0)
    wait_out(base + _APW - 1, 1)

    @pl.when(wid == _NW - 1)
    def _():
        pltpu.sync_copy(tail_flat, s_out[0].at[pl.ds(0, _ATAIL * _E)])
        pltpu.sync_copy(s_out[0].at[pl.ds(0, _ATAIL * _E)],
                        scratch.at[pl.ds(_NFULL * _AU * _E, _ATAIL * _E)])


_R2C = 1664
_R2CPW = _TOTAL // (_NW * _R2C)
_NBUF = 3


def _rows_body(idx_hbm, table_hbm, out_hbm, *scratch):
    idx_bufs = scratch[0:_NBUF]
    row_bufs = scratch[_NBUF:2 * _NBUF]
    gsems = scratch[2 * _NBUF:3 * _NBUF]
    wsems = scratch[3 * _NBUF:4 * _NBUF]
    wid = lax.axis_index("s") * _NC + lax.axis_index("c")

    def chunk_base(j):
        return (wid * _R2CPW + j) * _R2C

    def stage(j):
        b = j % _NBUF
        pltpu.sync_copy(idx_hbm.at[pl.ds(chunk_base(j), _R2C)], idx_bufs[b])
        pltpu.async_copy(table_hbm.at[idx_bufs[b]], row_bufs[b], gsems[b])

    for j in range(_NBUF - 1):
        stage(j)

    for j in range(_R2CPW):
        b = j % _NBUF
        nj = j + _NBUF - 1
        if nj < _R2CPW:
            nb = nj % _NBUF
            if nj >= _NBUF:
                pltpu.make_async_copy(
                    row_bufs[nb],
                    out_hbm.at[pl.ds(chunk_base(nj - _NBUF), _R2C)],
                    wsems[nb],
                ).wait()
            stage(nj)
        pltpu.make_async_copy(
            table_hbm.at[idx_bufs[b]], row_bufs[b], gsems[b]
        ).wait()
        pltpu.async_copy(
            row_bufs[b], out_hbm.at[pl.ds(chunk_base(j), _R2C)], wsems[b]
        )

    for j in range(max(_R2CPW - _NBUF, 0), _R2CPW):
        b = j % _NBUF
        pltpu.make_async_copy(
            row_bufs[b], out_hbm.at[pl.ds(chunk_base(j), _R2C)], wsems[b]
        ).wait()


def _gather_body(idx_hbm, table2d, out_hbm, idx0, idx1, rows0, rows1,
                 ov0, ov1, g_sem0, g_sem1, o_sem0, o_sem1):
    wid = lax.axis_index("s") * _NC + lax.axis_index("c")
    i1 = lax.iota(jnp.int32, 16)
    idx_v = (idx0, idx1)
    rows_v = (rows0, rows1)
    ov = (ov0, ov1)
    g_sem = (g_sem0, g_sem1)
    o_sem = (o_sem0, o_sem1)

    def chunk_of(lq):
        g = wid * _CPW + jnp.minimum(lq, _CPW - 1)
        f = g // _HPF
        h = g % _HPF
        return f * _B + h * _C, f * (2 * _B * 8) + h * (_C * 8)

    def fetch(lq, b):
        src, _ = chunk_of(lq)
        pltpu.sync_copy(idx_hbm.at[pl.ds(src, _C)], idx_v[b])
        pltpu.async_copy(table2d.at[idx_v[b]], rows_v[b], g_sem[b])

    def wait_gather(b):
        pltpu.make_async_copy(table2d.at[idx_v[b]], rows_v[b], g_sem[b]).wait()

    def out_copy(lq, b, dt):
        _, dst = chunk_of(lq)
        return pltpu.make_async_copy(
            ov[b].at[pl.ds(dt * (_C * 8), _C * 8)],
            out_hbm.at[pl.ds(dst + dt * (_B * 8), _C * 8)],
            o_sem[b],
        )

    def do_chunk(lq, s, b):
        wait_gather(b)

        @pl.when(s > 0)
        def _():
            out_copy(lq - 2, b, 0).wait()
            out_copy(lq - 2, b, 1).wait()

        for dt in range(2):
            for ds in range(8):
                d = dt * 8 + ds
                ci = jnp.full((16,), d, jnp.int32)
                for btl in range(_C // 128):
                    for k in range(8):
                        v = plsc.load_gather(
                            rows_v[b], [i1 + (btl * 128 + 16 * k), ci]
                        )
                        ov[b][pl.ds(dt * (_C * 8) + btl * 1024 + ds * 128
                                    + 16 * k, 16)] = v
        out_copy(lq, b, 0).start()
        out_copy(lq, b, 1).start()
        fetch(lq + 2, b)

    fetch(0, 0)
    fetch(1, 1)

    def step(s, carry):
        do_chunk(2 * s, s, 0)
        do_chunk(2 * s + 1, s, 1)
        return carry

    lax.fori_loop(0, _CPW // 2, step, 0)
    wait_gather(0)
    wait_gather(1)
    out_copy(_CPW - 2, 0, 0).wait()
    out_copy(_CPW - 2, 0, 1).wait()
    out_copy(_CPW - 1, 1, 0).wait()
    out_copy(_CPW - 1, 1, 1).wait()


@jax.jit
def _embed(inputs, table):
    mesh = plsc.VectorSubcoreMesh(core_axis_name="c", subcore_axis_name="s")
    transpose_call = functools.partial(
        pl.kernel,
        mesh=mesh,
        out_type=jax.ShapeDtypeStruct((_V * _E,), jnp.float32),
        compiler_params=pltpu.CompilerParams(needs_layout_passes=False),
        scratch_types=(
            [pltpu.VMEM((_E, _AU), jnp.float32) for _ in range(2)]
            + [pltpu.VMEM((_AU * _E,), jnp.float32) for _ in range(2)]
            + [pltpu.SemaphoreType.DMA for _ in range(4)]
        ),
    )(_transpose_body)
    tail_flat = table[_NFULL * _AU:, :].reshape(_ATAIL * _E)
    scratch = transpose_call(table.T, tail_flat)

    gather_call = functools.partial(
        pl.kernel,
        mesh=mesh,
        out_type=jax.ShapeDtypeStruct((_F * _E * _B,), jnp.float32),
        compiler_params=pltpu.CompilerParams(
            use_tc_tiling_on_sc=False, needs_layout_passes=False
        ),
        scratch_types=(
            [pltpu.VMEM((_C,), jnp.int32) for _ in range(2)]
            + [pltpu.VMEM((_C, _E), jnp.float32) for _ in range(2)]
            + [pltpu.VMEM((_C * _E,), jnp.float32) for _ in range(2)]
            + [pltpu.SemaphoreType.DMA for _ in range(4)]
        ),
    )(_gather_body)
    out_flat = gather_call(inputs.T.reshape(_TOTAL),
                           scratch.reshape(_V, _E))
    out5 = out_flat.reshape(_F, 2, _B // 128, 8, 128)
    return out5.transpose(2, 4, 0, 1, 3).reshape(_B, _F, _E)


def kernel(inputs, table):
    return _embed(inputs, table)
